# Initial kernel scaffold; baseline (speedup 1.0000x reference)
import functools

import jax
import jax.numpy as jnp
from jax import lax
from jax.experimental import pallas as pl
from jax.experimental.pallas import tpu as pltpu
from jax.experimental.pallas import tpu_sc as plsc

BS = 4096
P = 100
V = 10000
D = 64

_NC = 2
_NS = 16
_NW = _NC * _NS
_CHUNK = (2 * BS) // _NW
_L = 16


def _sc_gather_body(x_hbm, f_hbm, table_hbm, out_hbm, f_v, xi_v, xg_v, ti_v,
                    rows_v, sem):
    wid = lax.axis_index("s") * _NC + lax.axis_index("c")
    base = wid * _CHUNK
    b0 = lax.rem(wid, _NS) * _CHUNK

    pltpu.sync_copy(f_hbm.at[pl.ds(base, _CHUNK)], f_v)

    for t in range(_CHUNK // _L):
        lanes = lax.iota(jnp.int32, _L) + t * _L
        f16 = f_v[pl.ds(t * _L, _L)]
        xi_v[pl.ds(t * _L, _L)] = (b0 + lanes) * (2 * P) + f16

    pltpu.async_copy(x_hbm.at[xi_v], xg_v, sem).wait()

    for t in range(_CHUNK // _L):
        f16 = f_v[pl.ds(t * _L, _L)]
        xg16 = xg_v[pl.ds(t * _L, _L)]
        ti_v[pl.ds(t * _L, _L)] = xg16 + f16 * V

    pltpu.async_copy(table_hbm.at[ti_v], rows_v, sem).wait()
    pltpu.sync_copy(rows_v, out_hbm.at[pl.ds(base, _CHUNK)])


_sc_gather = functools.partial(
    pl.kernel,
    mesh=plsc.VectorSubcoreMesh(core_axis_name="c", subcore_axis_name="s"),
    out_type=jax.ShapeDtypeStruct((2 * BS, D), jnp.float32),
    scratch_types=[
        pltpu.VMEM((_CHUNK,), jnp.int32),
        pltpu.VMEM((_CHUNK,), jnp.int32),
        pltpu.VMEM((_CHUNK,), jnp.int32),
        pltpu.VMEM((_CHUNK,), jnp.int32),
        pltpu.VMEM((_CHUNK, D), jnp.float32),
        pltpu.SemaphoreType.DMA,
    ],
)(_sc_gather_body)


_BT = 512
_MP = 104


def _tc_body(rows_ref, feat_ref, mark_ref, w_ref, b_ref, ff_ref, mask_ref,
             pred_ref):
    f = feat_ref[...]
    iota = lax.broadcasted_iota(jnp.int32, (_BT, _MP), 1)
    hi = jax.lax.Precision.HIGHEST
    dn = (((1,), (0,)), ((), ()))
    oh0 = (iota == f[:, 0:1] + 1).astype(jnp.float32)
    oh1 = (iota == f[:, 1:2] + 1).astype(jnp.float32)
    m0 = lax.dot_general(oh0, mark_ref[...], dn, precision=hi,
                         preferred_element_type=jnp.float32)
    m1 = lax.dot_general(oh1, mark_ref[...], dn, precision=hi,
                         preferred_element_type=jnp.float32)
    ff0 = rows_ref[0] + m0
    ff1 = rows_ref[1] + m1
    ff_ref[0] = ff0
    ff_ref[1] = ff1
    wmat = w_ref[...]
    dnt = (((1,), (1,)), ((), ()))
    pred = lax.dot_general(ff0, wmat[:, :D], dnt, precision=hi,
                           preferred_element_type=jnp.float32)
    pred += lax.dot_general(ff1, wmat[:, D:], dnt, precision=hi,
                            preferred_element_type=jnp.float32)
    pred_ref[...] = pred + b_ref[...][None, :]
    mask_ref[...] = f == 0


_tc_finish = pl.pallas_call(
    _tc_body,
    grid=(BS // _BT,),
    in_specs=[
        pl.BlockSpec((2, _BT, D), lambda i: (0, i, 0)),
        pl.BlockSpec((_BT, 2), lambda i: (i, 0)),
        pl.BlockSpec((_MP, D), lambda i: (0, 0)),
        pl.BlockSpec((P, 2 * D), lambda i: (0, 0)),
        pl.BlockSpec((P,), lambda i: (0,)),
    ],
    out_specs=[
        pl.BlockSpec((2, _BT, D), lambda i: (0, i, 0)),
        pl.BlockSpec((_BT, 2), lambda i: (i, 0)),
        pl.BlockSpec((_BT, P), lambda i: (i, 0)),
    ],
    out_shape=[
        jax.ShapeDtypeStruct((2, BS, D), jnp.float32),
        jax.ShapeDtypeStruct((BS, 2), jnp.bool_),
        jax.ShapeDtypeStruct((BS, P), jnp.float32),
    ],
)


def kernel(x, features, table, mark_features, dummy_feature, W, b, idx_offset):
    del dummy_feature, idx_offset
    f_flat = features.T.reshape(-1)
    x_flat = x.reshape(-1)
    rows = _sc_gather(x_flat, f_flat, table)
    mark = mark_features.reshape(P + 1, D)
    mark = jnp.pad(mark, ((0, _MP - (P + 1)), (0, 0)))
    ff, mask, pred_n = _tc_finish(rows.reshape(2, BS, D), features, mark, W, b)
    return ff, mask, pred_n


# speedup vs baseline: 1.4540x; 1.4540x over previous
import functools

import jax
import jax.numpy as jnp
from jax import lax
from jax.experimental import pallas as pl
from jax.experimental.pallas import tpu as pltpu
from jax.experimental.pallas import tpu_sc as plsc

BS = 4096
P = 100
V = 10000
D = 64

_NC = 2
_NS = 16
_NW = _NC * _NS
_CHUNK = (2 * BS) // _NW
_L = 16


def _sc_gather_body(x_hbm, f_hbm, table_hbm, out_hbm, f_v, xi_v, xg_v, ti_v,
                    rows_v, sem):
    wid = lax.axis_index("s") * _NC + lax.axis_index("c")
    base = wid * _CHUNK
    b0 = lax.rem(wid, _NS) * _CHUNK

    pltpu.sync_copy(f_hbm.at[pl.ds(base, _CHUNK)], f_v)

    for t in range(_CHUNK // _L):
        lanes = lax.iota(jnp.int32, _L) + t * _L
        f16 = f_v[pl.ds(t * _L, _L)]
        xi_v[pl.ds(t * _L, _L)] = (b0 + lanes) * (2 * P) + f16

    pltpu.async_copy(x_hbm.at[xi_v], xg_v, sem).wait()

    for t in range(_CHUNK // _L):
        f16 = f_v[pl.ds(t * _L, _L)]
        xg16 = xg_v[pl.ds(t * _L, _L)]
        ti_v[pl.ds(t * _L, _L)] = xg16 + f16 * V

    pltpu.async_copy(table_hbm.at[ti_v], rows_v, sem).wait()
    pltpu.sync_copy(rows_v, out_hbm.at[pl.ds(base, _CHUNK)])


@functools.cache
def _sc_gather():
    return pl.kernel(
        _sc_gather_body,
        mesh=plsc.VectorSubcoreMesh(core_axis_name="c", subcore_axis_name="s"),
        compiler_params=pltpu.CompilerParams(use_tc_tiling_on_sc=False),
        out_type=jax.ShapeDtypeStruct((2 * BS, D), jnp.float32),
        scratch_types=[
            pltpu.VMEM((_CHUNK,), jnp.int32),
            pltpu.VMEM((_CHUNK,), jnp.int32),
            pltpu.VMEM((_CHUNK,), jnp.int32),
            pltpu.VMEM((_CHUNK,), jnp.int32),
            pltpu.VMEM((_CHUNK, D), jnp.float32),
            pltpu.SemaphoreType.DMA,
        ],
    )


_BT = 512
_MP = 104


def _tc_body(rows_ref, feat_ref, mark_ref, w_ref, b_ref, ff_ref, mask_ref,
             pred_ref):
    f = feat_ref[...]
    iota = lax.broadcasted_iota(jnp.int32, (_BT, _MP), 1)
    hi = jax.lax.Precision.HIGHEST
    dn = (((1,), (0,)), ((), ()))
    oh0 = (iota == f[:, 0:1] + 1).astype(jnp.float32)
    oh1 = (iota == f[:, 1:2] + 1).astype(jnp.float32)
    m0 = lax.dot_general(oh0, mark_ref[...], dn, precision=hi,
                         preferred_element_type=jnp.float32)
    m1 = lax.dot_general(oh1, mark_ref[...], dn, precision=hi,
                         preferred_element_type=jnp.float32)
    ff0 = rows_ref[0] + m0
    ff1 = rows_ref[1] + m1
    ff_ref[0] = ff0
    ff_ref[1] = ff1
    wmat = w_ref[...]
    dnt = (((1,), (1,)), ((), ()))
    pred = lax.dot_general(ff0, wmat[:, :D], dnt, precision=hi,
                           preferred_element_type=jnp.float32)
    pred += lax.dot_general(ff1, wmat[:, D:], dnt, precision=hi,
                            preferred_element_type=jnp.float32)
    pred_ref[...] = pred + b_ref[...][None, :]
    mask_ref[...] = f == 0


_tc_finish = pl.pallas_call(
    _tc_body,
    grid=(BS // _BT,),
    in_specs=[
        pl.BlockSpec((2, _BT, D), lambda i: (0, i, 0)),
        pl.BlockSpec((_BT, 2), lambda i: (i, 0)),
        pl.BlockSpec((_MP, D), lambda i: (0, 0)),
        pl.BlockSpec((P, 2 * D), lambda i: (0, 0)),
        pl.BlockSpec((P,), lambda i: (0,)),
    ],
    out_specs=[
        pl.BlockSpec((2, _BT, D), lambda i: (0, i, 0)),
        pl.BlockSpec((_BT, 2), lambda i: (i, 0)),
        pl.BlockSpec((_BT, P), lambda i: (i, 0)),
    ],
    out_shape=[
        jax.ShapeDtypeStruct((2, BS, D), jnp.float32),
        jax.ShapeDtypeStruct((BS, 2), jnp.bool_),
        jax.ShapeDtypeStruct((BS, P), jnp.float32),
    ],
)


def kernel(x, features, table, mark_features, dummy_feature, W, b, idx_offset):
    del dummy_feature, idx_offset
    f_flat = features.T.reshape(-1)
    x_flat = x.reshape(-1)
    rows = _sc_gather()(x_flat, f_flat, table)
    mark = mark_features.reshape(P + 1, D)
    mark = jnp.pad(mark, ((0, _MP - (P + 1)), (0, 0)))
    ff, mask, pred_n = _tc_finish(rows.reshape(2, BS, D), features, mark, W, b)
    return ff, mask, pred_n
